# baseline (speedup 1.0000x reference)
import functools

import jax
import jax.numpy as jnp
from jax.experimental import pallas as pl

N = 64
ROWS = 16 * 512


def _band_mask():
    i = jax.lax.broadcasted_iota(jnp.int32, (N, N), 0)
    j = jax.lax.broadcasted_iota(jnp.int32, (N, N), 1)
    m = j - i
    g1 = (m >= 0) & (m <= 15)
    g2 = (m >= 17) & (m <= 31) & (m % 2 == 1) & (i % 2 == 0)
    g3 = (m >= 35) & (m <= 63) & (m % 4 == 3) & (i % 4 == 0)
    return g1 | g2 | g3


def _pool_kernel(x_ref, o_ref):
    xb = x_ref[0]
    R = xb.shape[0]
    neg = jnp.float32(-jnp.inf)
    i = jax.lax.broadcasted_iota(jnp.int32, (N, N), 0)
    j = jax.lax.broadcasted_iota(jnp.int32, (N, N), 1)
    t = jnp.where((j >= i)[None], xb[:, None, :], neg)
    s = 1
    while s < N:
        pad = jnp.full((R, N, s), neg, jnp.float32)
        shifted = jnp.concatenate([pad, t[:, :, : N - s]], axis=-1)
        t = jnp.maximum(t, shifted)
        s *= 2
    m = j - i
    g1 = (m >= 0) & (m <= 15)
    g2 = (m >= 17) & (m <= 31) & (m % 2 == 1) & (i % 2 == 0)
    g3 = (m >= 35) & (m <= 63) & (m % 4 == 3) & (i % 4 == 0)
    mask = (g1 | g2 | g3)[None]
    o_ref[0] = jnp.where(mask, t, jnp.float32(0.0))


@functools.partial(jax.jit, static_argnames=("rows_per_block",))
def _run(x, rows_per_block):
    B, D = x.shape[0], x.shape[1]
    grid = (B, D // rows_per_block)
    return pl.pallas_call(
        _pool_kernel,
        grid=grid,
        in_specs=[pl.BlockSpec((1, rows_per_block, N), lambda b, d: (b, d, 0))],
        out_specs=pl.BlockSpec(
            (1, rows_per_block, N, N), lambda b, d: (b, d, 0, 0)
        ),
        out_shape=jax.ShapeDtypeStruct((B, D, N, N), jnp.float32),
    )(x)


def kernel(x):
    return _run(x, 256)


# speedup vs baseline: 1.3370x; 1.3370x over previous
import functools

import jax
import jax.numpy as jnp
from jax.experimental import pallas as pl
from jax.experimental.pallas import tpu as pltpu

N = 64
RB = 256
SPLIT = 4


def _pool_kernel(x_ref, o_ref, scratch, sems):
    b = pl.program_id(0)
    d = pl.program_id(1)
    nd = pl.num_programs(1)
    k = b * nd + d
    nsteps = pl.num_programs(0) * nd
    buf = jax.lax.rem(k, 2)
    rows = RB // SPLIT

    def copies(which_buf, bb, dd):
        return [
            pltpu.make_async_copy(
                scratch.at[which_buf, pl.ds(q * rows, rows)],
                o_ref.at[bb, pl.ds(dd * RB + q * rows, rows)],
                sems.at[which_buf, q],
            )
            for q in range(SPLIT)
        ]

    @pl.when(k >= 2)
    def _wait_prev():
        for c in copies(buf, b, d):
            c.wait()

    neg = jnp.float32(-jnp.inf)
    p = jax.lax.broadcasted_iota(jnp.int32, (N // 2, 2 * N), 0)
    l = jax.lax.broadcasted_iota(jnp.int32, (N // 2, 2 * N), 1)
    i = p + (N // 2) * (l // N)
    j = l % N
    m = j - i
    g1 = (m >= 0) & (m <= 15)
    g2 = (m >= 17) & (m <= 31) & (m % 2 == 1) & (i % 2 == 0)
    g3 = (m >= 35) & (m <= 63) & (m % 4 == 3) & (i % 4 == 0)
    mask = (g1 | g2 | g3)[None]

    xb = x_ref[0]
    x2 = jnp.concatenate([xb, xb], axis=-1)
    t = jnp.where((j >= i)[None], x2[:, None, :], neg)
    s = 1
    while s < N:
        pad = jnp.full((RB, N // 2, s), neg, jnp.float32)
        shifted = jnp.concatenate([pad, t[:, :, : 2 * N - s]], axis=-1)
        t = jnp.maximum(t, jnp.where((j >= s)[None], shifted, neg))
        s *= 2
    out = jnp.where(mask, t, jnp.float32(0.0))
    scratch[buf, :, : N // 2, :] = out[:, :, :N]
    scratch[buf, :, N // 2 :, :] = out[:, :, N:]
    del out

    for c in copies(buf, b, d):
        c.start()

    @pl.when(k == nsteps - 1)
    def _drain():
        for c in copies(buf, b, d):
            c.wait()

    @pl.when((k == nsteps - 1) & (nsteps >= 2))
    def _drain_other():
        prev = k - 1
        pb = prev // nd
        pd = jax.lax.rem(prev, nd)
        for c in copies(1 - buf, pb, pd):
            c.wait()


@functools.partial(jax.jit)
def _run(x):
    B, D = x.shape[0], x.shape[1]
    grid = (B, D // RB)
    return pl.pallas_call(
        _pool_kernel,
        grid=grid,
        in_specs=[pl.BlockSpec((1, RB, N), lambda b, d: (b, d, 0))],
        out_specs=pl.BlockSpec(memory_space=pl.ANY),
        out_shape=jax.ShapeDtypeStruct((B, D, N, N), jnp.float32),
        scratch_shapes=[
            pltpu.VMEM((2, RB, N, N), jnp.float32),
            pltpu.SemaphoreType.DMA((2, SPLIT)),
        ],
    )(x)


def kernel(x):
    return _run(x)
